# Initial kernel scaffold; baseline (speedup 1.0000x reference)
import jax
import jax.numpy as jnp
from jax.experimental import pallas as pl


def kernel(X):
    raise NotImplementedError("write your pallas kernel here")


# speedup vs baseline: 39.6570x; 39.6570x over previous
import jax
import jax.numpy as jnp
from jax.experimental import pallas as pl
from jax.experimental.pallas import tpu as pltpu

_ITERS = 16
_BLOCK_ROWS = 8


def _tsallis_block(x_ref, o_ref, xs_ref):
    x = x_ref[...]
    maxv = jnp.max(x, axis=1, keepdims=True)
    xs_ref[...] = (x - maxv) * 0.5

    lo0 = jnp.full_like(maxv, -1.0)
    hi0 = jnp.zeros_like(maxv)

    def body(_, carry):
        tau, lo, hi = carry
        r = jnp.maximum(xs_ref[...] - tau, 0.0)
        F = jnp.sum(r * r, axis=1, keepdims=True)
        H = jnp.sum(r, axis=1, keepdims=True)
        n = jnp.sum(jnp.where(r > 0.0, 1.0, 0.0), axis=1, keepdims=True)
        below = F >= 1.0
        lo = jnp.where(below, tau, lo)
        hi = jnp.where(below, hi, tau)
        disc = H * H - n * (F - 1.0)
        tq = tau + (H - jnp.sqrt(jnp.maximum(disc, 0.0))) / jnp.maximum(n, 1.0)
        ok = (disc >= 0.0) & (n > 0.0) & (tq > lo) & (tq < hi)
        tau = jnp.where(ok, tq, (lo + hi) * 0.5)
        return tau, lo, hi

    tau, _, _ = jax.lax.fori_loop(0, _ITERS, body, (lo0, lo0, hi0))
    r = jnp.maximum(xs_ref[...] - tau, 0.0)
    o_ref[...] = r * r


def kernel(X):
    R, L = X.shape
    return pl.pallas_call(
        _tsallis_block,
        grid=(R // _BLOCK_ROWS,),
        in_specs=[pl.BlockSpec((_BLOCK_ROWS, L), lambda i: (i, 0))],
        out_specs=pl.BlockSpec((_BLOCK_ROWS, L), lambda i: (i, 0)),
        out_shape=jax.ShapeDtypeStruct((R, L), jnp.float32),
        scratch_shapes=[pltpu.VMEM((_BLOCK_ROWS, L), jnp.float32)],
    )(X)


# speedup vs baseline: 69.7128x; 1.7579x over previous
import jax
import jax.numpy as jnp
from jax.experimental import pallas as pl
from jax.experimental.pallas import tpu as pltpu

_ITERS = 8
_BLOCK_ROWS = 16
_CH = 512


def _tsallis_block(x_ref, o_ref, xs_ref):
    L = x_ref.shape[1]

    macc = x_ref[:, 0:_CH]
    for c in range(_CH, L, _CH):
        macc = jnp.maximum(macc, x_ref[:, c:c + _CH])
    maxv = jnp.max(macc, axis=1, keepdims=True)

    for c in range(0, L, _CH):
        xs_ref[:, c:c + _CH] = (x_ref[:, c:c + _CH] - maxv) * 0.5

    lo0 = jnp.full_like(maxv, -1.0)
    hi0 = jnp.zeros_like(maxv)

    def body(_, carry):
        tau, lo, hi = carry
        fa = jnp.zeros((_BLOCK_ROWS, _CH), jnp.float32)
        ha = jnp.zeros((_BLOCK_ROWS, _CH), jnp.float32)
        na = jnp.zeros((_BLOCK_ROWS, _CH), jnp.float32)
        for c in range(0, L, _CH):
            r = jnp.maximum(xs_ref[:, c:c + _CH] - tau, 0.0)
            ha = ha + r
            fa = fa + r * r
            na = na + jnp.sign(r)
        F = jnp.sum(fa, axis=1, keepdims=True)
        H = jnp.sum(ha, axis=1, keepdims=True)
        n = jnp.sum(na, axis=1, keepdims=True)
        below = F >= 1.0
        lo = jnp.where(below, tau, lo)
        hi = jnp.where(below, hi, tau)
        disc = H * H - n * (F - 1.0)
        tq = tau + (H - jnp.sqrt(jnp.maximum(disc, 0.0))) / jnp.maximum(n, 1.0)
        ok = (disc >= 0.0) & (n > 0.0) & (tq >= lo) & (tq <= hi)
        tau = jnp.where(ok, tq, (lo + hi) * 0.5)
        return tau, lo, hi

    tau, _, _ = jax.lax.fori_loop(0, _ITERS, body, (lo0, lo0, hi0))

    for c in range(0, L, _CH):
        r = jnp.maximum(xs_ref[:, c:c + _CH] - tau, 0.0)
        o_ref[:, c:c + _CH] = r * r


def kernel(X):
    R, L = X.shape
    return pl.pallas_call(
        _tsallis_block,
        grid=(R // _BLOCK_ROWS,),
        in_specs=[pl.BlockSpec((_BLOCK_ROWS, L), lambda i: (i, 0))],
        out_specs=pl.BlockSpec((_BLOCK_ROWS, L), lambda i: (i, 0)),
        out_shape=jax.ShapeDtypeStruct((R, L), jnp.float32),
        scratch_shapes=[pltpu.VMEM((_BLOCK_ROWS, L), jnp.float32)],
        compiler_params=pltpu.CompilerParams(
            dimension_semantics=("parallel",)),
    )(X)


# speedup vs baseline: 81.3175x; 1.1665x over previous
import jax
import jax.numpy as jnp
from jax.experimental import pallas as pl
from jax.experimental.pallas import tpu as pltpu

_LOOP_ITERS = 7
_BLOCK_ROWS = 16
_CH = 256


def _solve(tau, lo, hi, F, H, n):
    below = F >= 1.0
    lo = jnp.where(below, tau, lo)
    hi = jnp.where(below, hi, tau)
    disc = H * H - n * (F - 1.0)
    tq = tau + (H - jnp.sqrt(jnp.maximum(disc, 0.0))) / jnp.maximum(n, 1.0)
    ok = (disc >= 0.0) & (n > 0.0) & (tq >= lo) & (tq <= hi)
    tau = jnp.where(ok, tq, (lo + hi) * 0.5)
    return tau, lo, hi


def _moments(fa, ha, na, r):
    ha = ha + r
    fa = fa + r * r
    na = na + jnp.where(r > 0.0, 1.0, 0.0)
    return fa, ha, na


def _tsallis_block(x_ref, o_ref, xs_ref):
    L = x_ref.shape[1]
    zeros = jnp.zeros((_BLOCK_ROWS, _CH), jnp.float32)

    macc = x_ref[:, 0:_CH]
    for c in range(_CH, L, _CH):
        macc = jnp.maximum(macc, x_ref[:, c:c + _CH])
    maxv = jnp.max(macc, axis=1, keepdims=True)

    fa, ha, na = zeros, zeros, zeros
    for c in range(0, L, _CH):
        xs = (x_ref[:, c:c + _CH] - maxv) * 0.5
        xs_ref[:, c:c + _CH] = xs
        fa, ha, na = _moments(fa, ha, na, jnp.maximum(xs + 1.0, 0.0))
    F = jnp.sum(fa, axis=1, keepdims=True)
    H = jnp.sum(ha, axis=1, keepdims=True)
    n = jnp.sum(na, axis=1, keepdims=True)

    lo0 = jnp.full_like(maxv, -1.0)
    hi0 = jnp.zeros_like(maxv)
    carry0 = _solve(lo0, lo0, hi0, F, H, n)

    def body(_, carry):
        tau, lo, hi = carry
        fa, ha, na = zeros, zeros, zeros
        for c in range(0, L, _CH):
            r = jnp.maximum(xs_ref[:, c:c + _CH] - tau, 0.0)
            fa, ha, na = _moments(fa, ha, na, r)
        F = jnp.sum(fa, axis=1, keepdims=True)
        H = jnp.sum(ha, axis=1, keepdims=True)
        n = jnp.sum(na, axis=1, keepdims=True)
        return _solve(tau, lo, hi, F, H, n)

    tau, _, _ = jax.lax.fori_loop(0, _LOOP_ITERS, body, carry0)

    for c in range(0, L, _CH):
        r = jnp.maximum(xs_ref[:, c:c + _CH] - tau, 0.0)
        o_ref[:, c:c + _CH] = r * r


def kernel(X):
    R, L = X.shape
    return pl.pallas_call(
        _tsallis_block,
        grid=(R // _BLOCK_ROWS,),
        in_specs=[pl.BlockSpec((_BLOCK_ROWS, L), lambda i: (i, 0))],
        out_specs=pl.BlockSpec((_BLOCK_ROWS, L), lambda i: (i, 0)),
        out_shape=jax.ShapeDtypeStruct((R, L), jnp.float32),
        scratch_shapes=[pltpu.VMEM((_BLOCK_ROWS, L), jnp.float32)],
        compiler_params=pltpu.CompilerParams(
            dimension_semantics=("parallel",)),
    )(X)


# speedup vs baseline: 90.3265x; 1.1108x over previous
import jax
import jax.numpy as jnp
from jax.experimental import pallas as pl
from jax.experimental.pallas import tpu as pltpu

_LOOP_ITERS = 8
_BLOCK_ROWS = 16
_CH = 256


def _solve(tau, lo, hi, F, H, n):
    below = F >= 1.0
    lo = jnp.where(below, tau, lo)
    hi = jnp.where(below, hi, tau)
    disc = H * H - n * (F - 1.0)
    tq = tau + (H - jnp.sqrt(jnp.maximum(disc, 0.0))) / jnp.maximum(n, 1.0)
    ok = (disc >= 0.0) & (n > 0.0) & (tq >= lo) & (tq <= hi)
    tau = jnp.where(ok, tq, (lo + hi) * 0.5)
    return tau, lo, hi


def _tsallis_block(x_ref, o_ref, xs_ref):
    L = x_ref.shape[1]
    zeros = jnp.zeros((_BLOCK_ROWS, _CH), jnp.float32)

    macc = x_ref[:, 0:_CH]
    for c in range(_CH, L, _CH):
        macc = jnp.maximum(macc, x_ref[:, c:c + _CH])
    maxv = jnp.max(macc, axis=1, keepdims=True)

    fa, ha, na = zeros, zeros, zeros
    for c in range(0, L, _CH):
        xs = (x_ref[:, c:c + _CH] - maxv) * 0.5
        xs_ref[:, c:c + _CH] = xs
        r = jnp.maximum(xs + 1.0, 0.0)
        ha = ha + r
        fa = fa + r * r
        na = na + jnp.where(r > 0.0, 1.0, 0.0)
    F = jnp.sum(fa, axis=1, keepdims=True)
    H = jnp.sum(ha, axis=1, keepdims=True)
    n = jnp.sum(na, axis=1, keepdims=True)

    lo0 = jnp.full_like(maxv, -1.0)
    hi0 = jnp.zeros_like(maxv)
    tau, lo, hi = _solve(lo0, lo0, hi0, F, H, n)

    def body(_, carry):
        tau, tau_p, H_p, lo, hi = carry
        fa, ha = zeros, zeros
        for c in range(0, L, _CH):
            r = jnp.maximum(xs_ref[:, c:c + _CH] - tau, 0.0)
            ha = ha + r
            fa = fa + r * r
        F = jnp.sum(fa, axis=1, keepdims=True)
        H = jnp.sum(ha, axis=1, keepdims=True)
        dt = tau - tau_p
        n = jnp.where(jnp.abs(dt) > 0.0, (H_p - H) / jnp.where(dt == 0.0, 1.0, dt), 1.0)
        tau_new, lo, hi = _solve(tau, lo, hi, F, H, n)
        return tau_new, tau, H, lo, hi

    tau, _, _, _, _ = jax.lax.fori_loop(
        0, _LOOP_ITERS, body, (tau, lo0, H, lo, hi))

    for c in range(0, L, _CH):
        r = jnp.maximum(xs_ref[:, c:c + _CH] - tau, 0.0)
        o_ref[:, c:c + _CH] = r * r


def kernel(X):
    R, L = X.shape
    return pl.pallas_call(
        _tsallis_block,
        grid=(R // _BLOCK_ROWS,),
        in_specs=[pl.BlockSpec((_BLOCK_ROWS, L), lambda i: (i, 0))],
        out_specs=pl.BlockSpec((_BLOCK_ROWS, L), lambda i: (i, 0)),
        out_shape=jax.ShapeDtypeStruct((R, L), jnp.float32),
        scratch_shapes=[pltpu.VMEM((_BLOCK_ROWS, L), jnp.float32)],
        compiler_params=pltpu.CompilerParams(
            dimension_semantics=("parallel",)),
    )(X)


# speedup vs baseline: 94.9228x; 1.0509x over previous
import jax
import jax.numpy as jnp
from jax.experimental import pallas as pl
from jax.experimental.pallas import tpu as pltpu

_LOOP_ITERS = 8
_BLOCK_ROWS = 16
_CH = 256


def _solve(tau, lo, hi, F, H, n):
    below = F >= 1.0
    lo = jnp.where(below, tau, lo)
    hi = jnp.where(below, hi, tau)
    disc = H * H - n * (F - 1.0)
    tq = tau + (H - jnp.sqrt(jnp.maximum(disc, 0.0))) / jnp.maximum(n, 1.0)
    ok = (disc >= 0.0) & (n > 0.0) & (tq >= lo) & (tq <= hi)
    tau = jnp.where(ok, tq, (lo + hi) * 0.5)
    return tau, lo, hi


def _tsallis_block(x_ref, o_ref, xs_ref):
    L = x_ref.shape[1]
    zeros = jnp.zeros((_BLOCK_ROWS, _CH), jnp.float32)

    macc = x_ref[:, 0:_CH]
    for c in range(_CH, L, _CH):
        macc = jnp.maximum(macc, x_ref[:, c:c + _CH])
    maxv = jnp.max(macc, axis=1, keepdims=True)

    fa, ha, na = zeros, zeros, zeros
    for c in range(0, L, _CH):
        xs = (x_ref[:, c:c + _CH] - maxv) * 0.5
        xs_ref[:, c:c + _CH] = xs
        r = jnp.maximum(xs + 1.0, 0.0)
        ha = ha + r
        fa = fa + r * r
        na = na + jnp.where(r > 0.0, 1.0, 0.0)
    F = jnp.sum(fa, axis=1, keepdims=True)
    H = jnp.sum(ha, axis=1, keepdims=True)
    n = jnp.sum(na, axis=1, keepdims=True)

    lo0 = jnp.full_like(maxv, -1.0)
    hi0 = jnp.zeros_like(maxv)
    tau, lo, hi = _solve(lo0, lo0, hi0, F, H, n)

    def body(_, carry):
        tau, tau_p, H_p, n_p, lo, hi = carry
        fa, ha = zeros, zeros
        for c in range(0, L, _CH):
            r = jnp.maximum(xs_ref[:, c:c + _CH] - tau, 0.0)
            ha = ha + r
            fa = fa + r * r
        F = jnp.sum(fa, axis=1, keepdims=True)
        H = jnp.sum(ha, axis=1, keepdims=True)
        dt = tau - tau_p
        n_est = (H_p - H) / jnp.where(dt == 0.0, 1.0, dt)
        use = (jnp.abs(dt) >= 3e-5) & (n_est > 0.0)
        n = jnp.where(use, n_est, n_p)
        tau_new, lo, hi = _solve(tau, lo, hi, F, H, n)
        return tau_new, tau, H, n, lo, hi

    tau, _, _, _, _, _ = jax.lax.fori_loop(
        0, _LOOP_ITERS, body, (tau, lo0, H, n, lo, hi))

    for c in range(0, L, _CH):
        r = jnp.maximum(xs_ref[:, c:c + _CH] - tau, 0.0)
        o_ref[:, c:c + _CH] = r * r


def kernel(X):
    R, L = X.shape
    return pl.pallas_call(
        _tsallis_block,
        grid=(R // _BLOCK_ROWS,),
        in_specs=[pl.BlockSpec((_BLOCK_ROWS, L), lambda i: (i, 0))],
        out_specs=pl.BlockSpec((_BLOCK_ROWS, L), lambda i: (i, 0)),
        out_shape=jax.ShapeDtypeStruct((R, L), jnp.float32),
        scratch_shapes=[pltpu.VMEM((_BLOCK_ROWS, L), jnp.float32)],
        compiler_params=pltpu.CompilerParams(
            dimension_semantics=("parallel",)),
    )(X)


# speedup vs baseline: 95.6608x; 1.0078x over previous
import jax
import jax.numpy as jnp
from jax.experimental import pallas as pl
from jax.experimental.pallas import tpu as pltpu

_LOOP_ITERS = 8
_BLOCK_ROWS = 16
_HALF = 8
_CH = 512


def _solve(tau, lo, hi, F, H, n):
    below = F >= 1.0
    lo = jnp.where(below, tau, lo)
    hi = jnp.where(below, hi, tau)
    disc = H * H - n * (F - 1.0)
    tq = tau + (H - jnp.sqrt(jnp.maximum(disc, 0.0))) / jnp.maximum(n, 1.0)
    ok = (disc >= 0.0) & (n > 0.0) & (tq >= lo) & (tq <= hi)
    tau = jnp.where(ok, tq, (lo + hi) * 0.5)
    return tau, lo, hi


def _estimate_n(tau, tau_p, H, H_p, n_p):
    dt = tau - tau_p
    n_est = (H_p - H) / jnp.where(dt == 0.0, 1.0, dt)
    use = (jnp.abs(dt) >= 3e-5) & (n_est > 0.0)
    return jnp.where(use, n_est, n_p)


def _sweep(xs_ref, rows, tau):
    L = xs_ref.shape[1]
    fa = jnp.zeros((_HALF, _CH), jnp.float32)
    ha = jnp.zeros((_HALF, _CH), jnp.float32)
    for c in range(0, L, _CH):
        r = jnp.maximum(xs_ref[rows, c:c + _CH] - tau, 0.0)
        ha = ha + r
        fa = fa + r * r
    F = jnp.sum(fa, axis=1, keepdims=True)
    H = jnp.sum(ha, axis=1, keepdims=True)
    return F, H


def _tsallis_block(x_ref, o_ref, xs_ref):
    L = x_ref.shape[1]
    rows_a = slice(0, _HALF)
    rows_b = slice(_HALF, _BLOCK_ROWS)

    macc = x_ref[:, 0:_CH]
    for c in range(_CH, L, _CH):
        macc = jnp.maximum(macc, x_ref[:, c:c + _CH])
    maxv = jnp.max(macc, axis=1, keepdims=True)

    fa = jnp.zeros((_BLOCK_ROWS, _CH), jnp.float32)
    ha = jnp.zeros_like(fa)
    na = jnp.zeros_like(fa)
    for c in range(0, L, _CH):
        xs = (x_ref[:, c:c + _CH] - maxv) * 0.5
        xs_ref[:, c:c + _CH] = xs
        r = jnp.maximum(xs + 1.0, 0.0)
        ha = ha + r
        fa = fa + r * r
        na = na + jnp.where(r > 0.0, 1.0, 0.0)
    F0 = jnp.sum(fa, axis=1, keepdims=True)
    H0 = jnp.sum(ha, axis=1, keepdims=True)
    n0 = jnp.sum(na, axis=1, keepdims=True)

    lo0 = jnp.full((_BLOCK_ROWS, 1), -1.0, jnp.float32)
    hi0 = jnp.zeros((_BLOCK_ROWS, 1), jnp.float32)

    za = F0[rows_a] * 0.0
    zb = F0[rows_b] * 0.0
    ta, la, ha_ = _solve(lo0[rows_a] + za, lo0[rows_a] + za, hi0[rows_a] + za,
                         F0[rows_a], H0[rows_a], n0[rows_a])
    sa = (ta, lo0[rows_a] + za, H0[rows_a], n0[rows_a], la, ha_)
    sb = (lo0[rows_b] + zb, lo0[rows_b] + zb, H0[rows_b], n0[rows_b],
          lo0[rows_b] + zb, hi0[rows_b] + zb, F0[rows_b], H0[rows_b])

    def body(_, carry):
        (ta, tpa, hpa, npa, la, hia), (tb, tpb, hpb, npb, lb, hib, fb, hb) = carry
        nb = _estimate_n(tb, tpb, hb, hpb, npb)
        tb_new, lb, hib = _solve(tb, lb, hib, fb, hb, nb)
        Fa, Ha = _sweep(xs_ref, rows_a, ta)
        na_ = _estimate_n(ta, tpa, Ha, hpa, npa)
        ta_new, la, hia = _solve(ta, la, hia, Fa, Ha, na_)
        Fb, Hb = _sweep(xs_ref, rows_b, tb_new)
        return ((ta_new, ta, Ha, na_, la, hia),
                (tb_new, tb, hb, nb, lb, hib, Fb, Hb))

    sa, sb = jax.lax.fori_loop(0, _LOOP_ITERS, body, (sa, sb))
    ta = sa[0]
    (tb, tpb, hpb, npb, lb, hib, fb, hb) = sb
    nb = _estimate_n(tb, tpb, hb, hpb, npb)
    tb, _, _ = _solve(tb, lb, hib, fb, hb, nb)

    for c in range(0, L, _CH):
        r = jnp.maximum(xs_ref[rows_a, c:c + _CH] - ta, 0.0)
        o_ref[rows_a, c:c + _CH] = r * r
    for c in range(0, L, _CH):
        r = jnp.maximum(xs_ref[rows_b, c:c + _CH] - tb, 0.0)
        o_ref[rows_b, c:c + _CH] = r * r


def kernel(X):
    R, L = X.shape
    return pl.pallas_call(
        _tsallis_block,
        grid=(R // _BLOCK_ROWS,),
        in_specs=[pl.BlockSpec((_BLOCK_ROWS, L), lambda i: (i, 0))],
        out_specs=pl.BlockSpec((_BLOCK_ROWS, L), lambda i: (i, 0)),
        out_shape=jax.ShapeDtypeStruct((R, L), jnp.float32),
        scratch_shapes=[pltpu.VMEM((_BLOCK_ROWS, L), jnp.float32)],
        compiler_params=pltpu.CompilerParams(
            dimension_semantics=("parallel",)),
    )(X)


# speedup vs baseline: 104.2939x; 1.0902x over previous
import jax
import jax.numpy as jnp
from jax.experimental import pallas as pl
from jax.experimental.pallas import tpu as pltpu

_LOOP_ITERS = 7
_BLOCK_ROWS = 16
_HALF = 8
_CH = 512


def _solve(tau, lo, hi, F, H, n):
    below = F >= 1.0
    lo = jnp.where(below, tau, lo)
    hi = jnp.where(below, hi, tau)
    disc = H * H - n * (F - 1.0)
    tq = tau + (H - jnp.sqrt(jnp.maximum(disc, 0.0))) / jnp.maximum(n, 1.0)
    ok = (disc >= 0.0) & (n > 0.0) & (tq >= lo) & (tq <= hi)
    tau = jnp.where(ok, tq, (lo + hi) * 0.5)
    return tau, lo, hi


def _estimate_n(tau, tau_p, H, H_p, n_p):
    dt = tau - tau_p
    n_est = (H_p - H) / jnp.where(dt == 0.0, 1.0, dt)
    use = (jnp.abs(dt) >= 3e-5) & (n_est > 0.0)
    return jnp.where(use, n_est, n_p)


def _sweep(xs_ref, rows, tau):
    L = xs_ref.shape[1]
    fa = jnp.zeros((_HALF, _CH), jnp.float32)
    ha = jnp.zeros((_HALF, _CH), jnp.float32)
    for c in range(0, L, _CH):
        r = jnp.maximum(xs_ref[rows, c:c + _CH] - tau, 0.0)
        ha = ha + r
        fa = fa + r * r
    F = jnp.sum(fa, axis=1, keepdims=True)
    H = jnp.sum(ha, axis=1, keepdims=True)
    return F, H


def _tsallis_block(x_ref, o_ref, xs_ref):
    L = x_ref.shape[1]
    rows_a = slice(0, _HALF)
    rows_b = slice(_HALF, _BLOCK_ROWS)

    macc = x_ref[:, 0:_CH]
    for c in range(_CH, L, _CH):
        macc = jnp.maximum(macc, x_ref[:, c:c + _CH])
    maxv = jnp.max(macc, axis=1, keepdims=True)

    fa = jnp.zeros((_BLOCK_ROWS, _CH), jnp.float32)
    ha = jnp.zeros_like(fa)
    na = jnp.zeros_like(fa)
    for c in range(0, L, _CH):
        xs = (x_ref[:, c:c + _CH] - maxv) * 0.5
        xs_ref[:, c:c + _CH] = xs
        r = jnp.maximum(xs + 1.0, 0.0)
        ha = ha + r
        fa = fa + r * r
        na = na + jnp.where(r > 0.0, 1.0, 0.0)
    F0 = jnp.sum(fa, axis=1, keepdims=True)
    H0 = jnp.sum(ha, axis=1, keepdims=True)
    n0 = jnp.sum(na, axis=1, keepdims=True)

    lo0 = jnp.full((_BLOCK_ROWS, 1), -1.0, jnp.float32)
    hi0 = jnp.zeros((_BLOCK_ROWS, 1), jnp.float32)

    za = F0[rows_a] * 0.0
    zb = F0[rows_b] * 0.0
    ta, la, ha_ = _solve(lo0[rows_a] + za, lo0[rows_a] + za, hi0[rows_a] + za,
                         F0[rows_a], H0[rows_a], n0[rows_a])
    sa = (ta, lo0[rows_a] + za, H0[rows_a], n0[rows_a], la, ha_)
    sb = (lo0[rows_b] + zb, lo0[rows_b] + zb, H0[rows_b], n0[rows_b],
          lo0[rows_b] + zb, hi0[rows_b] + zb, F0[rows_b], H0[rows_b])

    def body(_, carry):
        (ta, tpa, hpa, npa, la, hia), (tb, tpb, hpb, npb, lb, hib, fb, hb) = carry
        nb = _estimate_n(tb, tpb, hb, hpb, npb)
        tb_new, lb, hib = _solve(tb, lb, hib, fb, hb, nb)
        Fa, Ha = _sweep(xs_ref, rows_a, ta)
        na_ = _estimate_n(ta, tpa, Ha, hpa, npa)
        ta_new, la, hia = _solve(ta, la, hia, Fa, Ha, na_)
        Fb, Hb = _sweep(xs_ref, rows_b, tb_new)
        return ((ta_new, ta, Ha, na_, la, hia),
                (tb_new, tb, hb, nb, lb, hib, Fb, Hb))

    sa, sb = jax.lax.fori_loop(0, _LOOP_ITERS, body, (sa, sb))
    ta = sa[0]
    (tb, tpb, hpb, npb, lb, hib, fb, hb) = sb
    nb = _estimate_n(tb, tpb, hb, hpb, npb)
    tb, _, _ = _solve(tb, lb, hib, fb, hb, nb)

    for c in range(0, L, _CH):
        r = jnp.maximum(xs_ref[rows_a, c:c + _CH] - ta, 0.0)
        o_ref[rows_a, c:c + _CH] = r * r
    for c in range(0, L, _CH):
        r = jnp.maximum(xs_ref[rows_b, c:c + _CH] - tb, 0.0)
        o_ref[rows_b, c:c + _CH] = r * r


def kernel(X):
    R, L = X.shape
    return pl.pallas_call(
        _tsallis_block,
        grid=(R // _BLOCK_ROWS,),
        in_specs=[pl.BlockSpec((_BLOCK_ROWS, L), lambda i: (i, 0))],
        out_specs=pl.BlockSpec((_BLOCK_ROWS, L), lambda i: (i, 0)),
        out_shape=jax.ShapeDtypeStruct((R, L), jnp.float32),
        scratch_shapes=[pltpu.VMEM((_BLOCK_ROWS, L), jnp.float32)],
        compiler_params=pltpu.CompilerParams(
            dimension_semantics=("parallel",)),
    )(X)


# speedup vs baseline: 105.9617x; 1.0160x over previous
import jax
import jax.numpy as jnp
from jax.experimental import pallas as pl
from jax.experimental.pallas import tpu as pltpu

_LOOP_ITERS = 7
_BLOCK_ROWS = 16
_HALF = 8
_CH = 1024


def _solve(tau, lo, hi, F, H, n):
    below = F >= 1.0
    lo = jnp.where(below, tau, lo)
    hi = jnp.where(below, hi, tau)
    disc = H * H - n * (F - 1.0)
    tq = tau + (H - jnp.sqrt(jnp.maximum(disc, 0.0))) / jnp.maximum(n, 1.0)
    ok = (disc >= 0.0) & (n > 0.0) & (tq >= lo) & (tq <= hi)
    tau = jnp.where(ok, tq, (lo + hi) * 0.5)
    return tau, lo, hi


def _estimate_n(tau, tau_p, H, H_p, n_p):
    dt = tau - tau_p
    n_est = (H_p - H) / jnp.where(dt == 0.0, 1.0, dt)
    use = (jnp.abs(dt) >= 3e-5) & (n_est > 0.0)
    return jnp.where(use, n_est, n_p)


def _sweep(xs_ref, rows, tau):
    L = xs_ref.shape[1]
    fa = jnp.zeros((_HALF, _CH), jnp.float32)
    ha = jnp.zeros((_HALF, _CH), jnp.float32)
    for c in range(0, L, _CH):
        r = jnp.maximum(xs_ref[rows, c:c + _CH] - tau, 0.0)
        ha = ha + r
        fa = fa + r * r
    F = jnp.sum(fa, axis=1, keepdims=True)
    H = jnp.sum(ha, axis=1, keepdims=True)
    return F, H


def _tsallis_block(x_ref, o_ref, xs_ref):
    L = x_ref.shape[1]
    rows_a = slice(0, _HALF)
    rows_b = slice(_HALF, _BLOCK_ROWS)

    macc = x_ref[:, 0:_CH]
    for c in range(_CH, L, _CH):
        macc = jnp.maximum(macc, x_ref[:, c:c + _CH])
    maxv = jnp.max(macc, axis=1, keepdims=True)

    fa = jnp.zeros((_BLOCK_ROWS, _CH), jnp.float32)
    ha = jnp.zeros_like(fa)
    na = jnp.zeros_like(fa)
    for c in range(0, L, _CH):
        xs = (x_ref[:, c:c + _CH] - maxv) * 0.5
        xs_ref[:, c:c + _CH] = xs
        r = jnp.maximum(xs + 1.0, 0.0)
        ha = ha + r
        fa = fa + r * r
        na = na + jnp.where(r > 0.0, 1.0, 0.0)
    F0 = jnp.sum(fa, axis=1, keepdims=True)
    H0 = jnp.sum(ha, axis=1, keepdims=True)
    n0 = jnp.sum(na, axis=1, keepdims=True)

    lo0 = jnp.full((_BLOCK_ROWS, 1), -1.0, jnp.float32)
    hi0 = jnp.zeros((_BLOCK_ROWS, 1), jnp.float32)

    za = F0[rows_a] * 0.0
    zb = F0[rows_b] * 0.0
    ta, la, ha_ = _solve(lo0[rows_a] + za, lo0[rows_a] + za, hi0[rows_a] + za,
                         F0[rows_a], H0[rows_a], n0[rows_a])
    sa = (ta, lo0[rows_a] + za, H0[rows_a], n0[rows_a], la, ha_)
    sb = (lo0[rows_b] + zb, lo0[rows_b] + zb, H0[rows_b], n0[rows_b],
          lo0[rows_b] + zb, hi0[rows_b] + zb, F0[rows_b], H0[rows_b])

    def body(_, carry):
        (ta, tpa, hpa, npa, la, hia), (tb, tpb, hpb, npb, lb, hib, fb, hb) = carry
        nb = _estimate_n(tb, tpb, hb, hpb, npb)
        tb_new, lb, hib = _solve(tb, lb, hib, fb, hb, nb)
        Fa, Ha = _sweep(xs_ref, rows_a, ta)
        na_ = _estimate_n(ta, tpa, Ha, hpa, npa)
        ta_new, la, hia = _solve(ta, la, hia, Fa, Ha, na_)
        Fb, Hb = _sweep(xs_ref, rows_b, tb_new)
        return ((ta_new, ta, Ha, na_, la, hia),
                (tb_new, tb, hb, nb, lb, hib, Fb, Hb))

    sa, sb = jax.lax.fori_loop(0, _LOOP_ITERS, body, (sa, sb))
    ta = sa[0]
    (tb, tpb, hpb, npb, lb, hib, fb, hb) = sb
    nb = _estimate_n(tb, tpb, hb, hpb, npb)
    tb, _, _ = _solve(tb, lb, hib, fb, hb, nb)

    for c in range(0, L, _CH):
        r = jnp.maximum(xs_ref[rows_a, c:c + _CH] - ta, 0.0)
        o_ref[rows_a, c:c + _CH] = r * r
    for c in range(0, L, _CH):
        r = jnp.maximum(xs_ref[rows_b, c:c + _CH] - tb, 0.0)
        o_ref[rows_b, c:c + _CH] = r * r


def kernel(X):
    R, L = X.shape
    return pl.pallas_call(
        _tsallis_block,
        grid=(R // _BLOCK_ROWS,),
        in_specs=[pl.BlockSpec((_BLOCK_ROWS, L), lambda i: (i, 0))],
        out_specs=pl.BlockSpec((_BLOCK_ROWS, L), lambda i: (i, 0)),
        out_shape=jax.ShapeDtypeStruct((R, L), jnp.float32),
        scratch_shapes=[pltpu.VMEM((_BLOCK_ROWS, L), jnp.float32)],
        compiler_params=pltpu.CompilerParams(
            dimension_semantics=("parallel",)),
    )(X)
